# baseline (speedup 1.0000x reference)
import functools

import jax
import jax.numpy as jnp
from jax import lax
from jax.experimental import pallas as pl
from jax.experimental.pallas import tpu as pltpu
from jax.experimental.pallas import tpu_sc as plsc

BATCH = 16384
DIM = 32
ROWS_PER_BLOCK = 4
BLOCK_W = ROWS_PER_BLOCK * DIM
L = 16
NC, NS = 2, 16
NW = NC * NS
BPW = BATCH // NW
CHUNK = 128
NCH = BPW // CHUNK

_mesh = plsc.VectorSubcoreMesh(core_axis_name="c", subcore_axis_name="s")


@functools.partial(
    pl.kernel,
    out_type=jax.ShapeDtypeStruct((BATCH,), jnp.float32),
    mesh=_mesh,
    compiler_params=pltpu.CompilerParams(needs_layout_passes=False,
                                         use_tc_tiling_on_sc=True),
    scratch_types=[
        pltpu.VMEM((NCH, CHUNK), jnp.int32),
        pltpu.VMEM((NCH, CHUNK), jnp.int32),
        pltpu.VMEM((NCH, CHUNK), jnp.int32),
        pltpu.VMEM((NCH, CHUNK), jnp.int32),
        pltpu.VMEM((NCH, CHUNK), jnp.int32),
        pltpu.VMEM((NCH, CHUNK), jnp.int32),
        pltpu.VMEM((CHUNK, BLOCK_W), jnp.float32),
        pltpu.VMEM((CHUNK, BLOCK_W), jnp.float32),
        pltpu.VMEM((CHUNK,), jnp.float32),
        pltpu.VMEM((CHUNK,), jnp.float32),
        pltpu.VMEM((BPW,), jnp.float32),
        pltpu.SemaphoreType.DMA,
    ],
)
def _mf_bpr(uid, iid, uemb, iemb, ubias, ibias, out,
            uraw, iraw, ublk, iblk, uoff, ioff,
            urows, irows, ub, ib, out_v, sem):
    wid = lax.axis_index("s") * NC + lax.axis_index("c")
    base = wid * BPW

    for j in range(NCH):
        pltpu.sync_copy(uid.at[pl.ds(base + j * CHUNK, CHUNK)], uraw.at[j])
        pltpu.sync_copy(iid.at[pl.ds(base + j * CHUNK, CHUNK)], iraw.at[j])
    for j in range(NCH):
        for k in range(CHUNK // L):
            s = pl.ds(k * L, L)
            u = uraw[j, s] - 1
            i = iraw[j, s] - 1
            uraw[j, s] = u
            iraw[j, s] = i
            ublk[j, s] = lax.shift_right_logical(u, 2)
            iblk[j, s] = lax.shift_right_logical(i, 2)
            uoff[j, s] = lax.shift_left(u & 3, 5)
            ioff[j, s] = lax.shift_left(i & 3, 5)

    lanes = jnp.arange(L, dtype=jnp.int32)

    for j in range(NCH):
        copies = [
            pltpu.async_copy(uemb.at[ublk.at[j]], urows, sem),
            pltpu.async_copy(iemb.at[iblk.at[j]], irows, sem),
            pltpu.async_copy(ubias.at[uraw.at[j]], ub, sem),
            pltpu.async_copy(ibias.at[iraw.at[j]], ib, sem),
        ]
        for c in copies:
            c.wait()

        def body(g, carry, j=j):
            l0 = g * L
            uo = uoff[j, pl.ds(l0, L)]
            io = ioff[j, pl.ds(l0, L)]
            acc = ub[pl.ds(l0, L)] + ib[pl.ds(l0, L)]
            for q in range(L):
                b = l0 + q
                m = lanes == q
                ou = jnp.max(jnp.where(m, uo, 0))
                oi = jnp.max(jnp.where(m, io, 0))
                p = (urows[b, pl.ds(ou, L)] * irows[b, pl.ds(oi, L)]
                     + urows[b, pl.ds(ou + L, L)] * irows[b, pl.ds(oi + L, L)])
                acc = acc + jnp.where(m, jnp.sum(p), 0.0)
            out_v[pl.ds(j * CHUNK + l0, L)] = acc
            return carry

        lax.fori_loop(0, CHUNK // L, body, 0)

    pltpu.sync_copy(out_v, out.at[pl.ds(base, BPW)])


def kernel(user_id, item_id, user_embedding, item_embedding, user_bias, item_bias):
    uemb = user_embedding.reshape(-1, BLOCK_W)
    iemb = item_embedding.reshape(-1, BLOCK_W)
    return _mf_bpr(user_id, item_id, uemb, iemb,
                   user_bias.reshape(-1), item_bias.reshape(-1))
